# baseline (speedup 1.0000x reference)
import functools

import jax
import jax.numpy as jnp
from jax import lax
from jax.experimental import pallas as pl
from jax.experimental.pallas import tpu as pltpu

N, C_IN, C_OUT, HEADS, G = 100000, 128, 128, 1, 1024
B = 2048
NB = -(-N // B)
NPAD = NB * B
W = 64
NEG = -1e30


def _kern(bases_ref, oks_ref, x_ref, batch_ref, wg1_ref, wg2_ref,
          w1_ref, b1_ref, w2_ref, b2_ref, out_ref, m_scr, d_scr, acc_scr):
    i = pl.program_id(0)

    @pl.when(i == 0)
    def _():
        m_scr[...] = jnp.full((G, 1), NEG, jnp.float32)
        d_scr[...] = jnp.zeros((G, 1), jnp.float32)
        acc_scr[...] = jnp.zeros((G, C_OUT), jnp.float32)

    x = x_ref[...]
    a1 = jnp.maximum(jnp.dot(x, wg1_ref[...],
                             preferred_element_type=jnp.float32), 0.0)
    alphaT = lax.dot_general(wg2_ref[...], a1, (((0,), (1,)), ((), ())),
                             preferred_element_type=jnp.float32)
    u = jnp.maximum(jnp.dot(x.astype(jnp.bfloat16), w1_ref[...],
                            preferred_element_type=jnp.float32)
                    + b1_ref[...], 0.0)
    ub = u.astype(jnp.bfloat16)
    bm = jnp.max(alphaT)
    e_row = jnp.exp(alphaT - bm)
    batch_row = batch_ref[0]

    def upd(base, w):
        iot = lax.broadcasted_iota(jnp.int32, (w, B), 0) + base
        wm = jnp.where(iot == batch_row, e_row, 0.0)
        part_d = jnp.sum(wm, axis=1, keepdims=True)
        part_a = jnp.dot(wm.astype(jnp.bfloat16), ub,
                         preferred_element_type=jnp.float32)
        m_old = m_scr[pl.ds(base, w), :]
        m_new = jnp.maximum(m_old, bm)
        c_old = jnp.exp(m_old - m_new)
        c_new = jnp.exp(bm - m_new)
        d_scr[pl.ds(base, w), :] = (d_scr[pl.ds(base, w), :] * c_old
                                    + part_d * c_new)
        acc_scr[pl.ds(base, w), :] = (acc_scr[pl.ds(base, w), :] * c_old
                                      + part_a * c_new)
        m_scr[pl.ds(base, w), :] = m_new

    ok = oks_ref[i] != 0

    @pl.when(ok)
    def _():
        upd(bases_ref[i], W)

    @pl.when(jnp.logical_not(ok))
    def _():
        upd(0, G)

    @pl.when(i == NB - 1)
    def _():
        d = d_scr[...]
        dsafe = d + 1e-16
        out_ref[...] = (jnp.dot(acc_scr[...], w2_ref[...],
                                preferred_element_type=jnp.float32) / dsafe
                        + b2_ref[...] * (d / dsafe))


@functools.partial(jax.jit, static_argnames=("interpret",))
def _run(x, batch, Wg1, Wg2, W1, b1, W2, b2, interpret=False):
    batch = batch.astype(jnp.int32)
    xp = jnp.pad(x, ((0, NPAD - N), (0, 0)))
    bp = jnp.pad(batch, (0, NPAD - N), constant_values=G)
    batch_r = bp.reshape(NB, 1, B)

    r = jnp.arange(NB)
    first = batch[r * B]
    last = batch[jnp.minimum((r + 1) * B - 1, N - 1)]
    bases = jnp.minimum(first - (first % 8), G - W).astype(jnp.int32)
    oks = (last < bases + W).astype(jnp.int32)

    smem = pl.BlockSpec(memory_space=pltpu.SMEM)
    out = pl.pallas_call(
        _kern,
        grid=(NB,),
        in_specs=[
            smem, smem,
            pl.BlockSpec((B, C_IN), lambda i: (i, 0)),
            pl.BlockSpec((1, 1, B), lambda i: (i, 0, 0)),
            pl.BlockSpec((C_IN, C_IN), lambda i: (0, 0)),
            pl.BlockSpec((C_IN, 1), lambda i: (0, 0)),
            pl.BlockSpec((C_IN, C_OUT), lambda i: (0, 0)),
            pl.BlockSpec((1, C_OUT), lambda i: (0, 0)),
            pl.BlockSpec((C_OUT, C_OUT), lambda i: (0, 0)),
            pl.BlockSpec((1, C_OUT), lambda i: (0, 0)),
        ],
        out_specs=pl.BlockSpec((G, C_OUT), lambda i: (0, 0)),
        out_shape=jax.ShapeDtypeStruct((G, C_OUT), jnp.float32),
        scratch_shapes=[
            pltpu.VMEM((G, 1), jnp.float32),
            pltpu.VMEM((G, 1), jnp.float32),
            pltpu.VMEM((G, C_OUT), jnp.float32),
        ],
        compiler_params=pltpu.CompilerParams(
            dimension_semantics=("arbitrary",)),
        interpret=interpret,
    )(bases, oks, xp, batch_r, Wg1, Wg2,
      W1.astype(jnp.bfloat16), b1.reshape(1, C_OUT),
      W2, b2.reshape(1, C_OUT))

    return out.reshape(G, C_OUT, HEADS)


def kernel(x, batch, Wg1, Wg2, W1, b1, W2, b2):
    return _run(x, batch, Wg1, Wg2, W1, b1, W2, b2)


# speedup vs baseline: 1.2149x; 1.2149x over previous
import functools

import jax
import jax.numpy as jnp
from jax import lax
from jax.experimental import pallas as pl
from jax.experimental.pallas import tpu as pltpu

N, C_IN, C_OUT, HEADS, G = 100000, 128, 128, 1, 1024
B = 4096
NB = -(-N // B)
NPAD = NB * B
W = 96
NEG = -1e30


def _kern(bases_ref, oks_ref, x_ref, batch_ref, wg1_ref, wg2_ref,
          w1_ref, b1_ref, w2_ref, b2_ref, out_ref, m_scr, d_scr, acc_scr):
    i = pl.program_id(0)

    @pl.when(i == 0)
    def _():
        m_scr[...] = jnp.full((G, 1), NEG, jnp.float32)
        d_scr[...] = jnp.zeros((G, 1), jnp.float32)
        acc_scr[...] = jnp.zeros((G, C_OUT), jnp.float32)

    x = x_ref[...]
    a1 = jnp.maximum(jnp.dot(x, wg1_ref[...],
                             preferred_element_type=jnp.float32), 0.0)
    alphaT = lax.dot_general(wg2_ref[...], a1, (((0,), (1,)), ((), ())),
                             preferred_element_type=jnp.float32)
    u = jnp.maximum(jnp.dot(x, w1_ref[...],
                            preferred_element_type=jnp.float32)
                    + b1_ref[...], 0.0)
    ub = u.astype(jnp.bfloat16)
    bm = jnp.max(alphaT)
    e_row = jnp.exp(alphaT - bm)
    batch_row = batch_ref[0]

    def upd(base, w):
        iot = lax.broadcasted_iota(jnp.int32, (w, B), 0) + base
        wm = jnp.where(iot == batch_row, e_row, 0.0)
        part_d = jnp.sum(wm, axis=1, keepdims=True)
        part_a = jnp.dot(wm.astype(jnp.bfloat16), ub,
                         preferred_element_type=jnp.float32)
        m_old = m_scr[pl.ds(base, w), :]
        m_new = jnp.maximum(m_old, bm)
        c_old = jnp.exp(m_old - m_new)
        c_new = jnp.exp(bm - m_new)
        d_scr[pl.ds(base, w), :] = (d_scr[pl.ds(base, w), :] * c_old
                                    + part_d * c_new)
        acc_scr[pl.ds(base, w), :] = (acc_scr[pl.ds(base, w), :] * c_old
                                      + part_a * c_new)
        m_scr[pl.ds(base, w), :] = m_new

    ok = oks_ref[i] != 0

    @pl.when(ok)
    def _():
        upd(bases_ref[i], W)

    @pl.when(jnp.logical_not(ok))
    def _():
        upd(0, G)

    @pl.when(i == NB - 1)
    def _():
        d = d_scr[...]
        dsafe = d + 1e-16
        out_ref[...] = (jnp.dot(acc_scr[...], w2_ref[...],
                                preferred_element_type=jnp.float32) / dsafe
                        + b2_ref[...] * (d / dsafe))


@functools.partial(jax.jit, static_argnames=("interpret",))
def _run(x, batch, Wg1, Wg2, W1, b1, W2, b2, interpret=False):
    batch = batch.astype(jnp.int32)
    xp = jnp.pad(x, ((0, NPAD - N), (0, 0)))
    bp = jnp.pad(batch, (0, NPAD - N), constant_values=G)
    batch_r = bp.reshape(NB, 1, B)

    r = jnp.arange(NB)
    first = batch[r * B]
    last = batch[jnp.minimum((r + 1) * B - 1, N - 1)]
    bases = jnp.minimum(first - (first % 8), G - W).astype(jnp.int32)
    oks = (last < bases + W).astype(jnp.int32)

    smem = pl.BlockSpec(memory_space=pltpu.SMEM)
    out = pl.pallas_call(
        _kern,
        grid=(NB,),
        in_specs=[
            smem, smem,
            pl.BlockSpec((B, C_IN), lambda i: (i, 0)),
            pl.BlockSpec((1, 1, B), lambda i: (i, 0, 0)),
            pl.BlockSpec((C_IN, C_IN), lambda i: (0, 0)),
            pl.BlockSpec((C_IN, 1), lambda i: (0, 0)),
            pl.BlockSpec((C_IN, C_OUT), lambda i: (0, 0)),
            pl.BlockSpec((1, C_OUT), lambda i: (0, 0)),
            pl.BlockSpec((C_OUT, C_OUT), lambda i: (0, 0)),
            pl.BlockSpec((1, C_OUT), lambda i: (0, 0)),
        ],
        out_specs=pl.BlockSpec((G, C_OUT), lambda i: (0, 0)),
        out_shape=jax.ShapeDtypeStruct((G, C_OUT), jnp.float32),
        scratch_shapes=[
            pltpu.VMEM((G, 1), jnp.float32),
            pltpu.VMEM((G, 1), jnp.float32),
            pltpu.VMEM((G, C_OUT), jnp.float32),
        ],
        compiler_params=pltpu.CompilerParams(
            dimension_semantics=("arbitrary",)),
        interpret=interpret,
    )(bases, oks, xp, batch_r, Wg1, Wg2,
      W1, b1.reshape(1, C_OUT), W2, b2.reshape(1, C_OUT))

    return out.reshape(G, C_OUT, HEADS)


def kernel(x, batch, Wg1, Wg2, W1, b1, W2, b2):
    return _run(x, batch, Wg1, Wg2, W1, b1, W2, b2)


# speedup vs baseline: 1.2517x; 1.0303x over previous
import functools

import jax
import jax.numpy as jnp
from jax import lax
from jax.experimental import pallas as pl
from jax.experimental.pallas import tpu as pltpu

N, C_IN, C_OUT, HEADS, G = 100000, 128, 128, 1, 1024
B = 8192
NB = -(-N // B)
NPAD = NB * B
W = 128
NEG = -1e30


def _kern(bases_ref, oks_ref, x_ref, batch_ref, wg1_ref, wg2_ref,
          w1_ref, b1_ref, w2_ref, b2_ref, out_ref, m_scr, d_scr, acc_scr):
    i = pl.program_id(0)

    @pl.when(i == 0)
    def _():
        m_scr[...] = jnp.full((G, 1), NEG, jnp.float32)
        d_scr[...] = jnp.zeros((G, 1), jnp.float32)
        acc_scr[...] = jnp.zeros((G, C_OUT), jnp.float32)

    x = x_ref[...]
    a1 = jnp.maximum(jnp.dot(x, wg1_ref[...],
                             preferred_element_type=jnp.float32), 0.0)
    alphaT = lax.dot_general(wg2_ref[...], a1, (((0,), (1,)), ((), ())),
                             preferred_element_type=jnp.float32)
    u = jnp.maximum(jnp.dot(x, w1_ref[...],
                            preferred_element_type=jnp.float32)
                    + b1_ref[...], 0.0)
    ub = u.astype(jnp.bfloat16)
    bm = jnp.max(alphaT)
    e_row = jnp.exp(alphaT - bm)
    batch_row = batch_ref[0]

    def upd(base, w):
        iot = lax.broadcasted_iota(jnp.int32, (w, B), 0) + base
        wm = jnp.where(iot == batch_row, e_row, 0.0)
        part_d = jnp.sum(wm, axis=1, keepdims=True)
        part_a = jnp.dot(wm.astype(jnp.bfloat16), ub,
                         preferred_element_type=jnp.float32)
        m_old = m_scr[pl.ds(base, w), :]
        m_new = jnp.maximum(m_old, bm)
        c_old = jnp.exp(m_old - m_new)
        c_new = jnp.exp(bm - m_new)
        d_scr[pl.ds(base, w), :] = (d_scr[pl.ds(base, w), :] * c_old
                                    + part_d * c_new)
        acc_scr[pl.ds(base, w), :] = (acc_scr[pl.ds(base, w), :] * c_old
                                      + part_a * c_new)
        m_scr[pl.ds(base, w), :] = m_new

    ok = oks_ref[i] != 0

    @pl.when(ok)
    def _():
        upd(bases_ref[i], W)

    @pl.when(jnp.logical_not(ok))
    def _():
        upd(0, G)

    @pl.when(i == NB - 1)
    def _():
        d = d_scr[...]
        dsafe = d + 1e-16
        out_ref[...] = (jnp.dot(acc_scr[...], w2_ref[...],
                                preferred_element_type=jnp.float32) / dsafe
                        + b2_ref[...] * (d / dsafe))


@functools.partial(jax.jit, static_argnames=("interpret",))
def _run(x, batch, Wg1, Wg2, W1, b1, W2, b2, interpret=False):
    batch = batch.astype(jnp.int32)
    xp = jnp.pad(x, ((0, NPAD - N), (0, 0)))
    bp = jnp.pad(batch, (0, NPAD - N), constant_values=G)
    batch_r = bp.reshape(NB, 1, B)

    r = jnp.arange(NB)
    first = batch[r * B]
    last = batch[jnp.minimum((r + 1) * B - 1, N - 1)]
    bases = jnp.minimum(first - (first % 8), G - W).astype(jnp.int32)
    oks = (last < bases + W).astype(jnp.int32)

    smem = pl.BlockSpec(memory_space=pltpu.SMEM)
    out = pl.pallas_call(
        _kern,
        grid=(NB,),
        in_specs=[
            smem, smem,
            pl.BlockSpec((B, C_IN), lambda i: (i, 0)),
            pl.BlockSpec((1, 1, B), lambda i: (i, 0, 0)),
            pl.BlockSpec((C_IN, C_IN), lambda i: (0, 0)),
            pl.BlockSpec((C_IN, 1), lambda i: (0, 0)),
            pl.BlockSpec((C_IN, C_OUT), lambda i: (0, 0)),
            pl.BlockSpec((1, C_OUT), lambda i: (0, 0)),
            pl.BlockSpec((C_OUT, C_OUT), lambda i: (0, 0)),
            pl.BlockSpec((1, C_OUT), lambda i: (0, 0)),
        ],
        out_specs=pl.BlockSpec((G, C_OUT), lambda i: (0, 0)),
        out_shape=jax.ShapeDtypeStruct((G, C_OUT), jnp.float32),
        scratch_shapes=[
            pltpu.VMEM((G, 1), jnp.float32),
            pltpu.VMEM((G, 1), jnp.float32),
            pltpu.VMEM((G, C_OUT), jnp.float32),
        ],
        compiler_params=pltpu.CompilerParams(
            dimension_semantics=("arbitrary",)),
        interpret=interpret,
    )(bases, oks, xp, batch_r, Wg1, Wg2,
      W1, b1.reshape(1, C_OUT), W2, b2.reshape(1, C_OUT))

    return out.reshape(G, C_OUT, HEADS)


def kernel(x, batch, Wg1, Wg2, W1, b1, W2, b2):
    return _run(x, batch, Wg1, Wg2, W1, b1, W2, b2)
